# baseline (speedup 1.0000x reference)
import functools

import jax
import jax.numpy as jnp
from jax.experimental import pallas as pl
from jax.experimental.pallas import tpu as pltpu

N = 4096
HGC = 128
BLKG = 256
BLKA = 256
BLKL = 512
BLKC = 512
THRESH = 1.6

_SEQ1 = pltpu.CompilerParams(dimension_semantics=("arbitrary",))


def _full_spec(shape):
    ndim = len(shape)
    return pl.BlockSpec(shape, lambda i, _n=ndim: (0,) * _n)


def _gram_kernel(znf_ref, znfT_ref, znm_ref, znmT_ref, df_ref, dm_ref):
    i = pl.program_id(0)
    gf = jax.lax.dot_general(znf_ref[...], znfT_ref[...],
                             (((1,), (0,)), ((), ())),
                             preferred_element_type=jnp.float32)
    gm = jax.lax.dot_general(znm_ref[...], znmT_ref[...],
                             (((1,), (0,)), ((), ())),
                             preferred_element_type=jnp.float32)
    row = jax.lax.broadcasted_iota(jnp.int32, (BLKG, N), 0) + i * BLKG
    col = jax.lax.broadcasted_iota(jnp.int32, (BLKG, N), 1)
    ondiag = row == col
    df_ref[...] = jnp.where(ondiag, 0.0, 1.0 - gf)
    dm_ref[...] = jnp.where(ondiag, 0.0, 1.0 - gm)


def _adj_kernel(denf_ref, denm_ref, df_ref, dm_ref, a_ref, deg_ref):
    i = pl.program_id(0)
    df = df_ref[...]
    dm = dm_ref[...]
    adj = (jnp.exp(-(df * df) / denf_ref[0, 0])
           + jnp.exp(-(dm * dm) / denm_ref[0, 0]))
    row = jax.lax.broadcasted_iota(jnp.int32, (BLKA, N), 0) + i * BLKA
    col = jax.lax.broadcasted_iota(jnp.int32, (BLKA, N), 1)
    keep = (adj > THRESH) & (row != col)
    a = jnp.where(keep, adj, 0.0)
    a_ref[...] = a
    deg_ref[0, 0, :] = jnp.sum(a, axis=1)


def _lhat_kernel(dinv_col_ref, dinv_row_ref, a_ref, l_ref):
    i = pl.program_id(0)
    dcol = dinv_col_ref[pl.ds(i * BLKL, BLKL), :]
    l_ref[...] = -((dcol * a_ref[...]) * dinv_row_ref[...])


def _tx1_kernel(x_ref, l_ref, tx1_ref):
    tx1_ref[...] = jnp.dot(l_ref[...], x_ref[...],
                           preferred_element_type=jnp.float32)


def _cheb_out_kernel(has_mlp, x_ref, tx1_ref, w0_ref, w1_ref, w2_ref, *rest):
    if has_mlp:
        (cw1_ref, cb1_ref, g_ref, b_ref, cw2_ref, cb2_ref,
         l_ref, out_ref) = rest
    else:
        l_ref, out_ref = rest
    i = pl.program_id(0)
    x_blk = x_ref[pl.ds(i * BLKC, BLKC), :]
    tx1_blk = tx1_ref[pl.ds(i * BLKC, BLKC), :]
    y2 = jnp.dot(l_ref[...], tx1_ref[...], preferred_element_type=jnp.float32)
    tx2 = 2.0 * y2 - x_blk
    o = (jnp.dot(x_blk, w0_ref[...], preferred_element_type=jnp.float32)
         + jnp.dot(tx1_blk, w1_ref[...], preferred_element_type=jnp.float32)
         + jnp.dot(tx2, w2_ref[...], preferred_element_type=jnp.float32))
    r = jnp.maximum(o, 0.0)
    if has_mlp:
        h = jnp.dot(r, cw1_ref[...], preferred_element_type=jnp.float32) \
            + cb1_ref[...]
        h = jnp.maximum(h, 0.0)
        h = h / jnp.sqrt(jnp.float32(1.0 + 1e-5)) * g_ref[...] + b_ref[...]
        out_ref[...] = (jnp.dot(h, cw2_ref[...],
                                preferred_element_type=jnp.float32)
                        + cb2_ref[...])
    else:
        out_ref[...] = r


def _cheb_call(lhat, x, w0, w1, w2, mlp=None):
    d = x.shape[1]
    nblk = N // BLKC
    tx1 = pl.pallas_call(
        _tx1_kernel,
        grid=(nblk,),
        in_specs=[
            _full_spec((N, d)),
            pl.BlockSpec((BLKC, N), lambda i: (i, 0)),
        ],
        out_specs=pl.BlockSpec((BLKC, d), lambda i: (i, 0)),
        out_shape=jax.ShapeDtypeStruct((N, d), jnp.float32),
        compiler_params=_SEQ1,
    )(x, lhat)

    in_specs = [
        _full_spec((N, d)),
        _full_spec((N, d)),
        _full_spec((d, HGC)),
        _full_spec((d, HGC)),
        _full_spec((d, HGC)),
    ]
    args = [x, tx1, w0, w1, w2]
    if mlp is not None:
        cw1, cb1, g, b, cw2, cb2 = mlp
        in_specs += [
            _full_spec((HGC, 256)),
            _full_spec((1, 256)),
            _full_spec((1, 256)),
            _full_spec((1, 256)),
            _full_spec((256, 2)),
            _full_spec((1, 2)),
        ]
        args += [cw1, cb1, g, b, cw2, cb2]
        out_d = 2
    else:
        out_d = HGC
    in_specs.append(pl.BlockSpec((BLKC, N), lambda i: (i, 0)))
    args.append(lhat)
    return pl.pallas_call(
        functools.partial(_cheb_out_kernel, mlp is not None),
        grid=(nblk,),
        in_specs=in_specs,
        out_specs=pl.BlockSpec((BLKC, out_d), lambda i: (i, 0)),
        out_shape=jax.ShapeDtypeStruct((N, out_d), jnp.float32),
        compiler_params=_SEQ1,
    )(*args)


def kernel(x1, x2, x3, W0_0, W0_1, W0_2, W1_0, W1_1, W1_2, W2_0, W2_1, W2_2,
           cls_w1, cls_b1, bn_g, bn_b, cls_w2, cls_b2):
    fused = jnp.concatenate([x1, x2], axis=1)
    zcf = fused - jnp.mean(fused, axis=1, keepdims=True)
    znf = zcf / jnp.linalg.norm(zcf, axis=1, keepdims=True)
    zcm = x3 - jnp.mean(x3, axis=1, keepdims=True)
    znm = zcm / jnp.linalg.norm(zcm, axis=1, keepdims=True)
    znfT = znf.T
    znmT = znm.T
    dmeta = x3.shape[1]
    dfeat = fused.shape[1]

    nblk_g = N // BLKG
    df, dm = pl.pallas_call(
        _gram_kernel,
        grid=(nblk_g,),
        in_specs=[
            pl.BlockSpec((BLKG, dfeat), lambda i: (i, 0)),
            _full_spec((dfeat, N)),
            pl.BlockSpec((BLKG, dmeta), lambda i: (i, 0)),
            _full_spec((dmeta, N)),
        ],
        out_specs=[
            pl.BlockSpec((BLKG, N), lambda i: (i, 0)),
            pl.BlockSpec((BLKG, N), lambda i: (i, 0)),
        ],
        out_shape=[
            jax.ShapeDtypeStruct((N, N), jnp.float32),
            jax.ShapeDtypeStruct((N, N), jnp.float32),
        ],
        compiler_params=_SEQ1,
    )(znf, znfT, znm, znmT)

    sig = jnp.mean(df)
    msig = jnp.mean(dm)
    denf = (2.0 * sig ** 2).reshape(1, 1)
    denm = (2.0 * msig ** 2).reshape(1, 1)

    nblk_a = N // BLKA
    a, deg3 = pl.pallas_call(
        _adj_kernel,
        grid=(nblk_a,),
        in_specs=[
            _full_spec((1, 1)),
            _full_spec((1, 1)),
            pl.BlockSpec((BLKA, N), lambda i: (i, 0)),
            pl.BlockSpec((BLKA, N), lambda i: (i, 0)),
        ],
        out_specs=[
            pl.BlockSpec((BLKA, N), lambda i: (i, 0)),
            pl.BlockSpec((1, 1, BLKA), lambda i: (i, 0, 0)),
        ],
        out_shape=[
            jax.ShapeDtypeStruct((N, N), jnp.float32),
            jax.ShapeDtypeStruct((nblk_a, 1, BLKA), jnp.float32),
        ],
        compiler_params=_SEQ1,
    )(denf, denm, df, dm)

    deg = deg3.reshape(N)
    dinv = jnp.where(deg > 0, 1.0 / jnp.sqrt(jnp.where(deg > 0, deg, 1.0)),
                     0.0)
    dinv_col = dinv.reshape(N, 1)
    dinv_row = dinv.reshape(1, N)

    nblk_l = N // BLKL
    lhat = pl.pallas_call(
        _lhat_kernel,
        grid=(nblk_l,),
        in_specs=[
            _full_spec((N, 1)),
            _full_spec((1, N)),
            pl.BlockSpec((BLKL, N), lambda i: (i, 0)),
        ],
        out_specs=pl.BlockSpec((BLKL, N), lambda i: (i, 0)),
        out_shape=jax.ShapeDtypeStruct((N, N), jnp.float32),
        compiler_params=_SEQ1,
    )(dinv_col, dinv_row, a)

    x = _cheb_call(lhat, fused, W0_0, W0_1, W0_2)
    x = _cheb_call(lhat, x, W1_0, W1_1, W1_2)
    out = _cheb_call(lhat, x, W2_0, W2_1, W2_2,
                     mlp=(cls_w1, cls_b1.reshape(1, 256),
                          bn_g.reshape(1, 256), bn_b.reshape(1, 256),
                          cls_w2, cls_b2.reshape(1, 2)))
    return out


# speedup vs baseline: 1.1134x; 1.1134x over previous
import functools

import jax
import jax.numpy as jnp
from jax.experimental import pallas as pl
from jax.experimental.pallas import tpu as pltpu

N = 4096
HGC = 128
BLKG = 256
BLKA = 256
BLKC = 512
THRESH = 1.6

_PAR1 = pltpu.CompilerParams(dimension_semantics=("parallel",))


def _full_spec(shape):
    ndim = len(shape)
    return pl.BlockSpec(shape, lambda i, _n=ndim: (0,) * _n)


def _dists(znf, znfT, znm, znmT, i, blk):
    gf = jax.lax.dot_general(znf, znfT, (((1,), (0,)), ((), ())),
                             preferred_element_type=jnp.float32)
    gm = jax.lax.dot_general(znm, znmT, (((1,), (0,)), ((), ())),
                             preferred_element_type=jnp.float32)
    row = jax.lax.broadcasted_iota(jnp.int32, (blk, N), 0) + i * blk
    col = jax.lax.broadcasted_iota(jnp.int32, (blk, N), 1)
    ondiag = row == col
    df = jnp.where(ondiag, 0.0, 1.0 - gf)
    dm = jnp.where(ondiag, 0.0, 1.0 - gm)
    return df, dm, row, col


def _gram_kernel(znf_ref, znfT_ref, znm_ref, znmT_ref, df_ref, dm_ref):
    i = pl.program_id(0)
    df, dm, _, _ = _dists(znf_ref[...], znfT_ref[...], znm_ref[...],
                          znmT_ref[...], i, BLKG)
    df_ref[...] = df
    dm_ref[...] = dm


def _adj_kernel(denf_ref, denm_ref, znf_ref, znfT_ref, znm_ref, znmT_ref,
                a_ref, deg_ref):
    i = pl.program_id(0)
    df, dm, row, col = _dists(znf_ref[...], znfT_ref[...], znm_ref[...],
                              znmT_ref[...], i, BLKA)
    adj = (jnp.exp(-(df * df) / denf_ref[0, 0])
           + jnp.exp(-(dm * dm) / denm_ref[0, 0]))
    keep = (adj > THRESH) & (row != col)
    a = jnp.where(keep, adj, 0.0)
    a_ref[...] = a
    deg_ref[0, 0, :] = jnp.sum(a, axis=1)


def _tx1_kernel(dinv_col_ref, dinv_row_ref, x_ref, a_ref, tx1_ref):
    i = pl.program_id(0)
    dcol = dinv_col_ref[pl.ds(i * BLKC, BLKC), :]
    lblk = -((dcol * a_ref[...]) * dinv_row_ref[...])
    tx1_ref[...] = jnp.dot(lblk, x_ref[...],
                           preferred_element_type=jnp.float32)


def _cheb_out_kernel(has_mlp, dinv_col_ref, dinv_row_ref, x_ref, tx1_ref,
                     w0_ref, w1_ref, w2_ref, *rest):
    if has_mlp:
        (cw1_ref, cb1_ref, g_ref, b_ref, cw2_ref, cb2_ref,
         a_ref, out_ref) = rest
    else:
        a_ref, out_ref = rest
    i = pl.program_id(0)
    dcol = dinv_col_ref[pl.ds(i * BLKC, BLKC), :]
    lblk = -((dcol * a_ref[...]) * dinv_row_ref[...])
    x_blk = x_ref[pl.ds(i * BLKC, BLKC), :]
    tx1_blk = tx1_ref[pl.ds(i * BLKC, BLKC), :]
    y2 = jnp.dot(lblk, tx1_ref[...], preferred_element_type=jnp.float32)
    tx2 = 2.0 * y2 - x_blk
    o = (jnp.dot(x_blk, w0_ref[...], preferred_element_type=jnp.float32)
         + jnp.dot(tx1_blk, w1_ref[...], preferred_element_type=jnp.float32)
         + jnp.dot(tx2, w2_ref[...], preferred_element_type=jnp.float32))
    r = jnp.maximum(o, 0.0)
    if has_mlp:
        h = jnp.dot(r, cw1_ref[...], preferred_element_type=jnp.float32) \
            + cb1_ref[...]
        h = jnp.maximum(h, 0.0)
        h = h / jnp.sqrt(jnp.float32(1.0 + 1e-5)) * g_ref[...] + b_ref[...]
        out_ref[...] = (jnp.dot(h, cw2_ref[...],
                                preferred_element_type=jnp.float32)
                        + cb2_ref[...])
    else:
        out_ref[...] = r


def _cheb_call(a, dinv_col, dinv_row, x, w0, w1, w2, mlp=None):
    d = x.shape[1]
    nblk = N // BLKC
    tx1 = pl.pallas_call(
        _tx1_kernel,
        grid=(nblk,),
        in_specs=[
            _full_spec((N, 1)),
            _full_spec((1, N)),
            _full_spec((N, d)),
            pl.BlockSpec((BLKC, N), lambda i: (i, 0)),
        ],
        out_specs=pl.BlockSpec((BLKC, d), lambda i: (i, 0)),
        out_shape=jax.ShapeDtypeStruct((N, d), jnp.float32),
        compiler_params=_PAR1,
    )(dinv_col, dinv_row, x, a)

    in_specs = [
        _full_spec((N, 1)),
        _full_spec((1, N)),
        _full_spec((N, d)),
        _full_spec((N, d)),
        _full_spec((d, HGC)),
        _full_spec((d, HGC)),
        _full_spec((d, HGC)),
    ]
    args = [dinv_col, dinv_row, x, tx1, w0, w1, w2]
    if mlp is not None:
        cw1, cb1, g, b, cw2, cb2 = mlp
        in_specs += [
            _full_spec((HGC, 256)),
            _full_spec((1, 256)),
            _full_spec((1, 256)),
            _full_spec((1, 256)),
            _full_spec((256, 2)),
            _full_spec((1, 2)),
        ]
        args += [cw1, cb1, g, b, cw2, cb2]
        out_d = 2
    else:
        out_d = HGC
    in_specs.append(pl.BlockSpec((BLKC, N), lambda i: (i, 0)))
    args.append(a)
    return pl.pallas_call(
        functools.partial(_cheb_out_kernel, mlp is not None),
        grid=(nblk,),
        in_specs=in_specs,
        out_specs=pl.BlockSpec((BLKC, out_d), lambda i: (i, 0)),
        out_shape=jax.ShapeDtypeStruct((N, out_d), jnp.float32),
        compiler_params=_PAR1,
    )(*args)


def kernel(x1, x2, x3, W0_0, W0_1, W0_2, W1_0, W1_1, W1_2, W2_0, W2_1, W2_2,
           cls_w1, cls_b1, bn_g, bn_b, cls_w2, cls_b2):
    fused = jnp.concatenate([x1, x2], axis=1)
    zcf = fused - jnp.mean(fused, axis=1, keepdims=True)
    znf = zcf / jnp.linalg.norm(zcf, axis=1, keepdims=True)
    zcm = x3 - jnp.mean(x3, axis=1, keepdims=True)
    znm = zcm / jnp.linalg.norm(zcm, axis=1, keepdims=True)
    znfT = znf.T
    znmT = znm.T
    dmeta = x3.shape[1]
    dfeat = fused.shape[1]

    nblk_g = N // BLKG
    df, dm = pl.pallas_call(
        _gram_kernel,
        grid=(nblk_g,),
        in_specs=[
            pl.BlockSpec((BLKG, dfeat), lambda i: (i, 0)),
            _full_spec((dfeat, N)),
            pl.BlockSpec((BLKG, dmeta), lambda i: (i, 0)),
            _full_spec((dmeta, N)),
        ],
        out_specs=[
            pl.BlockSpec((BLKG, N), lambda i: (i, 0)),
            pl.BlockSpec((BLKG, N), lambda i: (i, 0)),
        ],
        out_shape=[
            jax.ShapeDtypeStruct((N, N), jnp.float32),
            jax.ShapeDtypeStruct((N, N), jnp.float32),
        ],
        compiler_params=_PAR1,
    )(znf, znfT, znm, znmT)

    sig = jnp.mean(df)
    msig = jnp.mean(dm)
    denf = (2.0 * sig ** 2).reshape(1, 1)
    denm = (2.0 * msig ** 2).reshape(1, 1)

    nblk_a = N // BLKA
    a, deg3 = pl.pallas_call(
        _adj_kernel,
        grid=(nblk_a,),
        in_specs=[
            _full_spec((1, 1)),
            _full_spec((1, 1)),
            pl.BlockSpec((BLKA, dfeat), lambda i: (i, 0)),
            _full_spec((dfeat, N)),
            pl.BlockSpec((BLKA, dmeta), lambda i: (i, 0)),
            _full_spec((dmeta, N)),
        ],
        out_specs=[
            pl.BlockSpec((BLKA, N), lambda i: (i, 0)),
            pl.BlockSpec((1, 1, BLKA), lambda i: (i, 0, 0)),
        ],
        out_shape=[
            jax.ShapeDtypeStruct((N, N), jnp.float32),
            jax.ShapeDtypeStruct((nblk_a, 1, BLKA), jnp.float32),
        ],
        compiler_params=_PAR1,
    )(denf, denm, znf, znfT, znm, znmT)

    deg = deg3.reshape(N)
    dinv = jnp.where(deg > 0, 1.0 / jnp.sqrt(jnp.where(deg > 0, deg, 1.0)),
                     0.0)
    dinv_col = dinv.reshape(N, 1)
    dinv_row = dinv.reshape(1, N)

    x = _cheb_call(a, dinv_col, dinv_row, fused, W0_0, W0_1, W0_2)
    x = _cheb_call(a, dinv_col, dinv_row, x, W1_0, W1_1, W1_2)
    out = _cheb_call(a, dinv_col, dinv_row, x, W2_0, W2_1, W2_2,
                     mlp=(cls_w1, cls_b1.reshape(1, 256),
                          bn_g.reshape(1, 256), bn_b.reshape(1, 256),
                          cls_w2, cls_b2.reshape(1, 2)))
    return out


# speedup vs baseline: 1.1145x; 1.0010x over previous
import functools

import jax
import jax.numpy as jnp
from jax.experimental import pallas as pl
from jax.experimental.pallas import tpu as pltpu

N = 4096
HGC = 128
BLKG = 256
BLKA = 256
BLKC = 512
THRESH = 1.6

_PAR1 = pltpu.CompilerParams(dimension_semantics=("parallel",))


def _full_spec(shape):
    ndim = len(shape)
    return pl.BlockSpec(shape, lambda i, _n=ndim: (0,) * _n)


def _dists(znf, znfT, znm, znmT, i, blk):
    gf = jax.lax.dot_general(znf, znfT, (((1,), (0,)), ((), ())),
                             preferred_element_type=jnp.float32)
    gm = jax.lax.dot_general(znm, znmT, (((1,), (0,)), ((), ())),
                             preferred_element_type=jnp.float32)
    row = jax.lax.broadcasted_iota(jnp.int32, (blk, N), 0) + i * blk
    col = jax.lax.broadcasted_iota(jnp.int32, (blk, N), 1)
    ondiag = row == col
    df = jnp.where(ondiag, 0.0, 1.0 - gf)
    dm = jnp.where(ondiag, 0.0, 1.0 - gm)
    return df, dm, row, col


def _gram_kernel(znf_ref, znfT_ref, znm_ref, znmT_ref, df_ref, dm_ref):
    i = pl.program_id(0)
    df, dm, _, _ = _dists(znf_ref[...], znfT_ref[...], znm_ref[...],
                          znmT_ref[...], i, BLKG)
    df_ref[...] = df
    dm_ref[...] = dm


def _adj_kernel(denf_ref, denm_ref, znf_ref, znfT_ref, znm_ref, znmT_ref,
                a_ref, deg_ref):
    i = pl.program_id(0)
    df, dm, row, col = _dists(znf_ref[...], znfT_ref[...], znm_ref[...],
                              znmT_ref[...], i, BLKA)
    adj = (jnp.exp(-(df * df) / denf_ref[0, 0])
           + jnp.exp(-(dm * dm) / denm_ref[0, 0]))
    keep = (adj > THRESH) & (row != col)
    a = jnp.where(keep, adj, 0.0)
    a_ref[...] = a
    deg_ref[0, 0, :] = jnp.sum(a, axis=1)


def _lhat_kernel(dinv_col_ref, dinv_row_ref, a_ref, l_ref):
    i = pl.program_id(0)
    dcol = dinv_col_ref[pl.ds(i * BLKA, BLKA), :]
    lblk = -((dcol * a_ref[...]) * dinv_row_ref[...])
    l_ref[...] = lblk.astype(jnp.bfloat16)


def _tx1_kernel(x_ref, l_ref, tx1_ref):
    tx1_ref[...] = jnp.dot(l_ref[...], x_ref[...],
                           preferred_element_type=jnp.float32)


def _cheb_out_kernel(has_mlp, x_ref, tx1_ref,
                     w0_ref, w1_ref, w2_ref, *rest):
    if has_mlp:
        (cw1_ref, cb1_ref, g_ref, b_ref, cw2_ref, cb2_ref,
         l_ref, out_ref) = rest
    else:
        l_ref, out_ref = rest
    i = pl.program_id(0)
    x_blk = x_ref[pl.ds(i * BLKC, BLKC), :]
    tx1_blk = tx1_ref[pl.ds(i * BLKC, BLKC), :]
    y2 = jnp.dot(l_ref[...], tx1_ref[...], preferred_element_type=jnp.float32)
    tx2 = 2.0 * y2 - x_blk
    o = (jnp.dot(x_blk, w0_ref[...], preferred_element_type=jnp.float32)
         + jnp.dot(tx1_blk, w1_ref[...], preferred_element_type=jnp.float32)
         + jnp.dot(tx2, w2_ref[...], preferred_element_type=jnp.float32))
    r = jnp.maximum(o, 0.0)
    if has_mlp:
        h = jnp.dot(r, cw1_ref[...], preferred_element_type=jnp.float32) \
            + cb1_ref[...]
        h = jnp.maximum(h, 0.0)
        h = h / jnp.sqrt(jnp.float32(1.0 + 1e-5)) * g_ref[...] + b_ref[...]
        out_ref[...] = (jnp.dot(h, cw2_ref[...],
                                preferred_element_type=jnp.float32)
                        + cb2_ref[...])
    else:
        out_ref[...] = r


def _cheb_call(lb, x, w0, w1, w2, mlp=None):
    d = x.shape[1]
    nblk = N // BLKC
    tx1 = pl.pallas_call(
        _tx1_kernel,
        grid=(nblk,),
        in_specs=[
            _full_spec((N, d)),
            pl.BlockSpec((BLKC, N), lambda i: (i, 0)),
        ],
        out_specs=pl.BlockSpec((BLKC, d), lambda i: (i, 0)),
        out_shape=jax.ShapeDtypeStruct((N, d), jnp.float32),
        compiler_params=_PAR1,
    )(x, lb)

    in_specs = [
        _full_spec((N, d)),
        _full_spec((N, d)),
        _full_spec((d, HGC)),
        _full_spec((d, HGC)),
        _full_spec((d, HGC)),
    ]
    args = [x, tx1, w0, w1, w2]
    if mlp is not None:
        cw1, cb1, g, b, cw2, cb2 = mlp
        in_specs += [
            _full_spec((HGC, 256)),
            _full_spec((1, 256)),
            _full_spec((1, 256)),
            _full_spec((1, 256)),
            _full_spec((256, 2)),
            _full_spec((1, 2)),
        ]
        args += [cw1, cb1, g, b, cw2, cb2]
        out_d = 2
    else:
        out_d = HGC
    in_specs.append(pl.BlockSpec((BLKC, N), lambda i: (i, 0)))
    args.append(lb)
    return pl.pallas_call(
        functools.partial(_cheb_out_kernel, mlp is not None),
        grid=(nblk,),
        in_specs=in_specs,
        out_specs=pl.BlockSpec((BLKC, out_d), lambda i: (i, 0)),
        out_shape=jax.ShapeDtypeStruct((N, out_d), jnp.float32),
        compiler_params=_PAR1,
    )(*args)


def kernel(x1, x2, x3, W0_0, W0_1, W0_2, W1_0, W1_1, W1_2, W2_0, W2_1, W2_2,
           cls_w1, cls_b1, bn_g, bn_b, cls_w2, cls_b2):
    fused = jnp.concatenate([x1, x2], axis=1)
    zcf = fused - jnp.mean(fused, axis=1, keepdims=True)
    znf = zcf / jnp.linalg.norm(zcf, axis=1, keepdims=True)
    zcm = x3 - jnp.mean(x3, axis=1, keepdims=True)
    znm = zcm / jnp.linalg.norm(zcm, axis=1, keepdims=True)
    znfT = znf.T
    znmT = znm.T
    dmeta = x3.shape[1]
    dfeat = fused.shape[1]

    nblk_g = N // BLKG
    df, dm = pl.pallas_call(
        _gram_kernel,
        grid=(nblk_g,),
        in_specs=[
            pl.BlockSpec((BLKG, dfeat), lambda i: (i, 0)),
            _full_spec((dfeat, N)),
            pl.BlockSpec((BLKG, dmeta), lambda i: (i, 0)),
            _full_spec((dmeta, N)),
        ],
        out_specs=[
            pl.BlockSpec((BLKG, N), lambda i: (i, 0)),
            pl.BlockSpec((BLKG, N), lambda i: (i, 0)),
        ],
        out_shape=[
            jax.ShapeDtypeStruct((N, N), jnp.float32),
            jax.ShapeDtypeStruct((N, N), jnp.float32),
        ],
        compiler_params=_PAR1,
    )(znf, znfT, znm, znmT)

    sig = jnp.mean(df)
    msig = jnp.mean(dm)
    denf = (2.0 * sig ** 2).reshape(1, 1)
    denm = (2.0 * msig ** 2).reshape(1, 1)

    nblk_a = N // BLKA
    a, deg3 = pl.pallas_call(
        _adj_kernel,
        grid=(nblk_a,),
        in_specs=[
            _full_spec((1, 1)),
            _full_spec((1, 1)),
            pl.BlockSpec((BLKA, dfeat), lambda i: (i, 0)),
            _full_spec((dfeat, N)),
            pl.BlockSpec((BLKA, dmeta), lambda i: (i, 0)),
            _full_spec((dmeta, N)),
        ],
        out_specs=[
            pl.BlockSpec((BLKA, N), lambda i: (i, 0)),
            pl.BlockSpec((1, 1, BLKA), lambda i: (i, 0, 0)),
        ],
        out_shape=[
            jax.ShapeDtypeStruct((N, N), jnp.float32),
            jax.ShapeDtypeStruct((nblk_a, 1, BLKA), jnp.float32),
        ],
        compiler_params=_PAR1,
    )(denf, denm, znf, znfT, znm, znmT)

    deg = deg3.reshape(N)
    dinv = jnp.where(deg > 0, 1.0 / jnp.sqrt(jnp.where(deg > 0, deg, 1.0)),
                     0.0)
    dinv_col = dinv.reshape(N, 1)
    dinv_row = dinv.reshape(1, N)

    lb = pl.pallas_call(
        _lhat_kernel,
        grid=(nblk_a,),
        in_specs=[
            _full_spec((N, 1)),
            _full_spec((1, N)),
            pl.BlockSpec((BLKA, N), lambda i: (i, 0)),
        ],
        out_specs=pl.BlockSpec((BLKA, N), lambda i: (i, 0)),
        out_shape=jax.ShapeDtypeStruct((N, N), jnp.bfloat16),
        compiler_params=_PAR1,
    )(dinv_col, dinv_row, a)

    x = _cheb_call(lb, fused, W0_0, W0_1, W0_2)
    x = _cheb_call(lb, x, W1_0, W1_1, W1_2)
    out = _cheb_call(lb, x, W2_0, W2_1, W2_2,
                     mlp=(cls_w1, cls_b1.reshape(1, 256),
                          bn_g.reshape(1, 256), bn_b.reshape(1, 256),
                          cls_w2, cls_b2.reshape(1, 2)))
    return out


# speedup vs baseline: 1.1532x; 1.0347x over previous
import functools

import jax
import jax.numpy as jnp
from jax.experimental import pallas as pl
from jax.experimental.pallas import tpu as pltpu

N = 4096
HGC = 128
BLKG = 256
BLKA = 256
BLKC = 512
THRESH = 1.6

_PAR1 = pltpu.CompilerParams(dimension_semantics=("parallel",))


def _full_spec(shape):
    ndim = len(shape)
    return pl.BlockSpec(shape, lambda i, _n=ndim: (0,) * _n)


def _dists(znf, znfT, znm, znmT, i, blk):
    gf = jax.lax.dot_general(znf, znfT, (((1,), (0,)), ((), ())),
                             preferred_element_type=jnp.float32)
    gm = jax.lax.dot_general(znm, znmT, (((1,), (0,)), ((), ())),
                             preferred_element_type=jnp.float32)
    row = jax.lax.broadcasted_iota(jnp.int32, (blk, N), 0) + i * blk
    col = jax.lax.broadcasted_iota(jnp.int32, (blk, N), 1)
    ondiag = row == col
    df = jnp.where(ondiag, 0.0, 1.0 - gf)
    dm = jnp.where(ondiag, 0.0, 1.0 - gm)
    return df, dm, row, col


def _gram_kernel(znf_ref, znfT_ref, znm_ref, znmT_ref, df_ref, dm_ref):
    i = pl.program_id(0)
    df, dm, _, _ = _dists(znf_ref[...], znfT_ref[...], znm_ref[...],
                          znmT_ref[...], i, BLKG)
    df_ref[...] = df
    dm_ref[...] = dm


def _adj_kernel(denf_ref, denm_ref, znf_ref, znfT_ref, znm_ref, znmT_ref,
                a_ref, deg_ref):
    i = pl.program_id(0)
    df, dm, row, col = _dists(znf_ref[...], znfT_ref[...], znm_ref[...],
                              znmT_ref[...], i, BLKA)
    adj = (jnp.exp(-(df * df) / denf_ref[0, 0])
           + jnp.exp(-(dm * dm) / denm_ref[0, 0]))
    keep = (adj > THRESH) & (row != col)
    a = jnp.where(keep, adj, 0.0)
    a_ref[...] = a
    deg_ref[0, 0, :] = jnp.sum(a, axis=1)


def _lhat_kernel(dinv_col_ref, dinv_row_ref, a_ref, l_ref):
    i = pl.program_id(0)
    dcol = dinv_col_ref[pl.ds(i * BLKA, BLKA), :]
    lblk = -((dcol * a_ref[...]) * dinv_row_ref[...])
    l_ref[...] = lblk.astype(jnp.bfloat16)


_NBLKC = N // BLKC


def _cheb_mega_kernel(xf_ref, w00, w01, w02, w10, w11, w12, w20, w21, w22,
                      cw1_ref, cb1_ref, g_ref, b_ref, cw2_ref, cb2_ref,
                      lb_ref, out_ref, t_s, xb_s):
    s = pl.program_id(0)
    ph = s // _NBLKC
    i = jax.lax.rem(s, _NBLKC)
    r0 = i * BLKC
    lb = lb_ref[...]

    @pl.when(ph == 0)
    def _tx1_l0():
        t_s[pl.ds(r0, BLKC), :] = jnp.dot(
            lb, xf_ref[...], preferred_element_type=jnp.float32)

    @pl.when(ph == 1)
    def _out_l0():
        x_blk = xf_ref[pl.ds(r0, BLKC), :]
        t_blk = t_s[pl.ds(r0, BLKC), :]
        y2 = jnp.dot(lb, t_s[...], preferred_element_type=jnp.float32)
        tx2 = 2.0 * y2 - x_blk
        o = (jnp.dot(x_blk, w00[...], preferred_element_type=jnp.float32)
             + jnp.dot(t_blk, w01[...], preferred_element_type=jnp.float32)
             + jnp.dot(tx2, w02[...], preferred_element_type=jnp.float32))
        xb_s[pl.ds(r0, BLKC), :] = jnp.maximum(o, 0.0)

    @pl.when(ph == 2)
    def _tx1_l1():
        t_s[pl.ds(r0, BLKC), 0:HGC] = jnp.dot(
            lb, xb_s[...], preferred_element_type=jnp.float32)

    @pl.when(ph == 3)
    def _out_l1():
        x_blk = xb_s[pl.ds(r0, BLKC), :]
        t_blk = t_s[pl.ds(r0, BLKC), 0:HGC]
        y2 = jnp.dot(lb, t_s[:, 0:HGC], preferred_element_type=jnp.float32)
        tx2 = 2.0 * y2 - x_blk
        o = (jnp.dot(x_blk, w10[...], preferred_element_type=jnp.float32)
             + jnp.dot(t_blk, w11[...], preferred_element_type=jnp.float32)
             + jnp.dot(tx2, w12[...], preferred_element_type=jnp.float32))
        xb_s[pl.ds(r0, BLKC), :] = jnp.maximum(o, 0.0)

    @pl.when(ph == 4)
    def _tx1_l2():
        t_s[pl.ds(r0, BLKC), 0:HGC] = jnp.dot(
            lb, xb_s[...], preferred_element_type=jnp.float32)

    @pl.when(ph == 5)
    def _out_l2_mlp():
        x_blk = xb_s[pl.ds(r0, BLKC), :]
        t_blk = t_s[pl.ds(r0, BLKC), 0:HGC]
        y2 = jnp.dot(lb, t_s[:, 0:HGC], preferred_element_type=jnp.float32)
        tx2 = 2.0 * y2 - x_blk
        o = (jnp.dot(x_blk, w20[...], preferred_element_type=jnp.float32)
             + jnp.dot(t_blk, w21[...], preferred_element_type=jnp.float32)
             + jnp.dot(tx2, w22[...], preferred_element_type=jnp.float32))
        r = jnp.maximum(o, 0.0)
        h = jnp.dot(r, cw1_ref[...], preferred_element_type=jnp.float32) \
            + cb1_ref[...]
        h = jnp.maximum(h, 0.0)
        h = h / jnp.sqrt(jnp.float32(1.0 + 1e-5)) * g_ref[...] + b_ref[...]
        out_ref[...] = (jnp.dot(h, cw2_ref[...],
                                preferred_element_type=jnp.float32)
                        + cb2_ref[...])


def _cheb_all_call(lb, fused, ws, mlp):
    dfeat = fused.shape[1]
    cw1, cb1, g, b, cw2, cb2 = mlp
    in_specs = [_full_spec((N, dfeat))]
    in_specs += [_full_spec((dfeat, HGC))] * 3
    in_specs += [_full_spec((HGC, HGC))] * 6
    in_specs += [
        _full_spec((HGC, 256)),
        _full_spec((1, 256)),
        _full_spec((1, 256)),
        _full_spec((1, 256)),
        _full_spec((256, 2)),
        _full_spec((1, 2)),
    ]
    in_specs.append(
        pl.BlockSpec((BLKC, N), lambda s: (jax.lax.rem(s, _NBLKC), 0)))
    return pl.pallas_call(
        _cheb_mega_kernel,
        grid=(6 * _NBLKC,),
        in_specs=in_specs,
        out_specs=pl.BlockSpec(
            (BLKC, 2), lambda s: (jnp.maximum(s - 5 * _NBLKC, 0), 0)),
        out_shape=jax.ShapeDtypeStruct((N, 2), jnp.float32),
        scratch_shapes=[
            pltpu.VMEM((N, 384), jnp.float32),
            pltpu.VMEM((N, HGC), jnp.float32),
        ],
        compiler_params=pltpu.CompilerParams(
            dimension_semantics=("arbitrary",)),
    )(fused, *ws, cw1, cb1, g, b, cw2, cb2, lb)


def kernel(x1, x2, x3, W0_0, W0_1, W0_2, W1_0, W1_1, W1_2, W2_0, W2_1, W2_2,
           cls_w1, cls_b1, bn_g, bn_b, cls_w2, cls_b2):
    fused = jnp.concatenate([x1, x2], axis=1)
    zcf = fused - jnp.mean(fused, axis=1, keepdims=True)
    znf = zcf / jnp.linalg.norm(zcf, axis=1, keepdims=True)
    zcm = x3 - jnp.mean(x3, axis=1, keepdims=True)
    znm = zcm / jnp.linalg.norm(zcm, axis=1, keepdims=True)
    znfT = znf.T
    znmT = znm.T
    dmeta = x3.shape[1]
    dfeat = fused.shape[1]

    nblk_g = N // BLKG
    df, dm = pl.pallas_call(
        _gram_kernel,
        grid=(nblk_g,),
        in_specs=[
            pl.BlockSpec((BLKG, dfeat), lambda i: (i, 0)),
            _full_spec((dfeat, N)),
            pl.BlockSpec((BLKG, dmeta), lambda i: (i, 0)),
            _full_spec((dmeta, N)),
        ],
        out_specs=[
            pl.BlockSpec((BLKG, N), lambda i: (i, 0)),
            pl.BlockSpec((BLKG, N), lambda i: (i, 0)),
        ],
        out_shape=[
            jax.ShapeDtypeStruct((N, N), jnp.float32),
            jax.ShapeDtypeStruct((N, N), jnp.float32),
        ],
        compiler_params=_PAR1,
    )(znf, znfT, znm, znmT)

    sig = jnp.mean(df)
    msig = jnp.mean(dm)
    denf = (2.0 * sig ** 2).reshape(1, 1)
    denm = (2.0 * msig ** 2).reshape(1, 1)

    nblk_a = N // BLKA
    a, deg3 = pl.pallas_call(
        _adj_kernel,
        grid=(nblk_a,),
        in_specs=[
            _full_spec((1, 1)),
            _full_spec((1, 1)),
            pl.BlockSpec((BLKA, dfeat), lambda i: (i, 0)),
            _full_spec((dfeat, N)),
            pl.BlockSpec((BLKA, dmeta), lambda i: (i, 0)),
            _full_spec((dmeta, N)),
        ],
        out_specs=[
            pl.BlockSpec((BLKA, N), lambda i: (i, 0)),
            pl.BlockSpec((1, 1, BLKA), lambda i: (i, 0, 0)),
        ],
        out_shape=[
            jax.ShapeDtypeStruct((N, N), jnp.float32),
            jax.ShapeDtypeStruct((nblk_a, 1, BLKA), jnp.float32),
        ],
        compiler_params=_PAR1,
    )(denf, denm, znf, znfT, znm, znmT)

    deg = deg3.reshape(N)
    dinv = jnp.where(deg > 0, 1.0 / jnp.sqrt(jnp.where(deg > 0, deg, 1.0)),
                     0.0)
    dinv_col = dinv.reshape(N, 1)
    dinv_row = dinv.reshape(1, N)

    lb = pl.pallas_call(
        _lhat_kernel,
        grid=(nblk_a,),
        in_specs=[
            _full_spec((N, 1)),
            _full_spec((1, N)),
            pl.BlockSpec((BLKA, N), lambda i: (i, 0)),
        ],
        out_specs=pl.BlockSpec((BLKA, N), lambda i: (i, 0)),
        out_shape=jax.ShapeDtypeStruct((N, N), jnp.bfloat16),
        compiler_params=_PAR1,
    )(dinv_col, dinv_row, a)

    out = _cheb_all_call(
        lb, fused,
        (W0_0, W0_1, W0_2, W1_0, W1_1, W1_2, W2_0, W2_1, W2_2),
        (cls_w1, cls_b1.reshape(1, 256), bn_g.reshape(1, 256),
         bn_b.reshape(1, 256), cls_w2, cls_b2.reshape(1, 2)))
    return out


# speedup vs baseline: 1.1847x; 1.0274x over previous
import functools

import jax
import jax.numpy as jnp
from jax.experimental import pallas as pl
from jax.experimental.pallas import tpu as pltpu

N = 4096
HGC = 128
BLKG = 256
BLKA = 256
BLKC = 512
THRESH = 1.6

_PAR1 = pltpu.CompilerParams(dimension_semantics=("parallel",))


def _full_spec(shape):
    ndim = len(shape)
    return pl.BlockSpec(shape, lambda i, _n=ndim: (0,) * _n)


def _dists(znf_blk, znf, znm_blk, znm, i, blk):
    gf = jax.lax.dot_general(znf_blk, znf, (((1,), (1,)), ((), ())),
                             preferred_element_type=jnp.float32)
    gm = jax.lax.dot_general(znm_blk, znm, (((1,), (1,)), ((), ())),
                             preferred_element_type=jnp.float32)
    row = jax.lax.broadcasted_iota(jnp.int32, (blk, N), 0) + i * blk
    col = jax.lax.broadcasted_iota(jnp.int32, (blk, N), 1)
    ondiag = row == col
    df = jnp.where(ondiag, 0.0, 1.0 - gf)
    dm = jnp.where(ondiag, 0.0, 1.0 - gm)
    return df, dm, row, col


def _gram_kernel(znf_ref, znm_ref, df_ref, dm_ref):
    i = pl.program_id(0)
    znf_blk = znf_ref[pl.ds(i * BLKG, BLKG), :]
    znm_blk = znm_ref[pl.ds(i * BLKG, BLKG), :]
    df, dm, _, _ = _dists(znf_blk, znf_ref[...], znm_blk, znm_ref[...],
                          i, BLKG)
    df_ref[...] = df
    dm_ref[...] = dm


def _adj_kernel(denf_ref, denm_ref, znf_ref, znm_ref, a_ref, deg_ref):
    i = pl.program_id(0)
    r0 = i * BLKA
    znf_blk = znf_ref[pl.ds(r0, BLKA), :]
    znm_blk = znm_ref[pl.ds(r0, BLKA), :]
    df, dm, row, col = _dists(znf_blk, znf_ref[...], znm_blk,
                              znm_ref[...], i, BLKA)
    adj = (jnp.exp(-(df * df) / denf_ref[0, 0])
           + jnp.exp(-(dm * dm) / denm_ref[0, 0]))
    keep = (adj > THRESH) & (row != col)
    a = jnp.where(keep, adj, 0.0)
    a_ref[...] = a
    deg_ref[0, 0, :] = jnp.sum(a, axis=1)


def _lhat_kernel(dinv_col_ref, dinv_row_ref, a_ref, l_ref):
    i = pl.program_id(0)
    dcol = dinv_col_ref[pl.ds(i * BLKA, BLKA), :]
    lblk = -((dcol * a_ref[...]) * dinv_row_ref[...])
    l_ref[...] = lblk.astype(jnp.bfloat16)


_NBLKC = N // BLKC


def _cheb_mega_kernel(xf_ref, w00, w01, w02, w10, w11, w12, w20, w21, w22,
                      cw1_ref, cb1_ref, g_ref, b_ref, cw2_ref, cb2_ref,
                      lb_ref, out_ref, t_s, xb_s):
    s = pl.program_id(0)
    ph = s // _NBLKC
    i = jax.lax.rem(s, _NBLKC)
    r0 = i * BLKC
    lb = lb_ref[...]

    @pl.when(ph == 0)
    def _tx1_l0():
        t_s[pl.ds(r0, BLKC), :] = jnp.dot(
            lb, xf_ref[...], preferred_element_type=jnp.float32)

    @pl.when(ph == 1)
    def _out_l0():
        x_blk = xf_ref[pl.ds(r0, BLKC), :]
        t_blk = t_s[pl.ds(r0, BLKC), :]
        y2 = jnp.dot(lb, t_s[...], preferred_element_type=jnp.float32)
        tx2 = 2.0 * y2 - x_blk
        o = (jnp.dot(x_blk, w00[...], preferred_element_type=jnp.float32)
             + jnp.dot(t_blk, w01[...], preferred_element_type=jnp.float32)
             + jnp.dot(tx2, w02[...], preferred_element_type=jnp.float32))
        xb_s[pl.ds(r0, BLKC), :] = jnp.maximum(o, 0.0)

    @pl.when(ph == 2)
    def _tx1_l1():
        t_s[pl.ds(r0, BLKC), 0:HGC] = jnp.dot(
            lb, xb_s[...], preferred_element_type=jnp.float32)

    @pl.when(ph == 3)
    def _out_l1():
        x_blk = xb_s[pl.ds(r0, BLKC), :]
        t_blk = t_s[pl.ds(r0, BLKC), 0:HGC]
        y2 = jnp.dot(lb, t_s[:, 0:HGC], preferred_element_type=jnp.float32)
        tx2 = 2.0 * y2 - x_blk
        o = (jnp.dot(x_blk, w10[...], preferred_element_type=jnp.float32)
             + jnp.dot(t_blk, w11[...], preferred_element_type=jnp.float32)
             + jnp.dot(tx2, w12[...], preferred_element_type=jnp.float32))
        xb_s[pl.ds(r0, BLKC), :] = jnp.maximum(o, 0.0)

    @pl.when(ph == 4)
    def _tx1_l2():
        t_s[pl.ds(r0, BLKC), 0:HGC] = jnp.dot(
            lb, xb_s[...], preferred_element_type=jnp.float32)

    @pl.when(ph == 5)
    def _out_l2_mlp():
        x_blk = xb_s[pl.ds(r0, BLKC), :]
        t_blk = t_s[pl.ds(r0, BLKC), 0:HGC]
        y2 = jnp.dot(lb, t_s[:, 0:HGC], preferred_element_type=jnp.float32)
        tx2 = 2.0 * y2 - x_blk
        o = (jnp.dot(x_blk, w20[...], preferred_element_type=jnp.float32)
             + jnp.dot(t_blk, w21[...], preferred_element_type=jnp.float32)
             + jnp.dot(tx2, w22[...], preferred_element_type=jnp.float32))
        r = jnp.maximum(o, 0.0)
        h = jnp.dot(r, cw1_ref[...], preferred_element_type=jnp.float32) \
            + cb1_ref[...]
        h = jnp.maximum(h, 0.0)
        h = h / jnp.sqrt(jnp.float32(1.0 + 1e-5)) * g_ref[...] + b_ref[...]
        out_ref[...] = (jnp.dot(h, cw2_ref[...],
                                preferred_element_type=jnp.float32)
                        + cb2_ref[...])


def _cheb_all_call(lb, fused, ws, mlp):
    dfeat = fused.shape[1]
    cw1, cb1, g, b, cw2, cb2 = mlp
    in_specs = [_full_spec((N, dfeat))]
    in_specs += [_full_spec((dfeat, HGC))] * 3
    in_specs += [_full_spec((HGC, HGC))] * 6
    in_specs += [
        _full_spec((HGC, 256)),
        _full_spec((1, 256)),
        _full_spec((1, 256)),
        _full_spec((1, 256)),
        _full_spec((256, 2)),
        _full_spec((1, 2)),
    ]
    in_specs.append(
        pl.BlockSpec((BLKC, N), lambda s: (jax.lax.rem(s, _NBLKC), 0)))
    return pl.pallas_call(
        _cheb_mega_kernel,
        grid=(6 * _NBLKC,),
        in_specs=in_specs,
        out_specs=pl.BlockSpec(
            (BLKC, 2), lambda s: (jnp.maximum(s - 5 * _NBLKC, 0), 0)),
        out_shape=jax.ShapeDtypeStruct((N, 2), jnp.float32),
        scratch_shapes=[
            pltpu.VMEM((N, 384), jnp.float32),
            pltpu.VMEM((N, HGC), jnp.float32),
        ],
        compiler_params=pltpu.CompilerParams(
            dimension_semantics=("arbitrary",)),
    )(fused, *ws, cw1, cb1, g, b, cw2, cb2, lb)


def kernel(x1, x2, x3, W0_0, W0_1, W0_2, W1_0, W1_1, W1_2, W2_0, W2_1, W2_2,
           cls_w1, cls_b1, bn_g, bn_b, cls_w2, cls_b2):
    fused = jnp.concatenate([x1, x2], axis=1)
    zcf = fused - jnp.mean(fused, axis=1, keepdims=True)
    znf = zcf / jnp.linalg.norm(zcf, axis=1, keepdims=True)
    zcm = x3 - jnp.mean(x3, axis=1, keepdims=True)
    znm = zcm / jnp.linalg.norm(zcm, axis=1, keepdims=True)
    dmeta = x3.shape[1]
    dfeat = fused.shape[1]

    nblk_g = N // BLKG
    df, dm = pl.pallas_call(
        _gram_kernel,
        grid=(nblk_g,),
        in_specs=[
            _full_spec((N, dfeat)),
            _full_spec((N, dmeta)),
        ],
        out_specs=[
            pl.BlockSpec((BLKG, N), lambda i: (i, 0)),
            pl.BlockSpec((BLKG, N), lambda i: (i, 0)),
        ],
        out_shape=[
            jax.ShapeDtypeStruct((N, N), jnp.float32),
            jax.ShapeDtypeStruct((N, N), jnp.float32),
        ],
        compiler_params=_PAR1,
    )(znf, znm)

    sig = jnp.mean(df)
    msig = jnp.mean(dm)
    denf = (2.0 * sig ** 2).reshape(1, 1)
    denm = (2.0 * msig ** 2).reshape(1, 1)

    nblk_a = N // BLKA
    a, deg3 = pl.pallas_call(
        _adj_kernel,
        grid=(nblk_a,),
        in_specs=[
            _full_spec((1, 1)),
            _full_spec((1, 1)),
            _full_spec((N, dfeat)),
            _full_spec((N, dmeta)),
        ],
        out_specs=[
            pl.BlockSpec((BLKA, N), lambda i: (i, 0)),
            pl.BlockSpec((1, 1, BLKA), lambda i: (i, 0, 0)),
        ],
        out_shape=[
            jax.ShapeDtypeStruct((N, N), jnp.float32),
            jax.ShapeDtypeStruct((nblk_a, 1, BLKA), jnp.float32),
        ],
        compiler_params=_PAR1,
    )(denf, denm, znf, znm)

    deg = deg3.reshape(N)
    dinv = jnp.where(deg > 0, 1.0 / jnp.sqrt(jnp.where(deg > 0, deg, 1.0)),
                     0.0)
    dinv_col = dinv.reshape(N, 1)
    dinv_row = dinv.reshape(1, N)

    lb = pl.pallas_call(
        _lhat_kernel,
        grid=(nblk_a,),
        in_specs=[
            _full_spec((N, 1)),
            _full_spec((1, N)),
            pl.BlockSpec((BLKA, N), lambda i: (i, 0)),
        ],
        out_specs=pl.BlockSpec((BLKA, N), lambda i: (i, 0)),
        out_shape=jax.ShapeDtypeStruct((N, N), jnp.bfloat16),
        compiler_params=_PAR1,
    )(dinv_col, dinv_row, a)

    out = _cheb_all_call(
        lb, fused,
        (W0_0, W0_1, W0_2, W1_0, W1_1, W1_2, W2_0, W2_1, W2_2),
        (cls_w1, cls_b1.reshape(1, 256), bn_g.reshape(1, 256),
         bn_b.reshape(1, 256), cls_w2, cls_b2.reshape(1, 2)))
    return out
